# Initial kernel scaffold; baseline (speedup 1.0000x reference)
import jax
import jax.numpy as jnp
from jax.experimental import pallas as pl


def kernel(x, neighbor_agg, h, neighbor):
    raise NotImplementedError("write your pallas kernel here")


# speedup vs baseline: 1.3700x; 1.3700x over previous
import functools

import jax
import jax.numpy as jnp
from jax.experimental import pallas as pl
from jax.experimental.pallas import tpu as pltpu

_ALPHA = 0.1
_BLOCK = 1000


def _appnp_block(x_ref, agg_ref, h_ref, nb_ref, x_out_ref, agg_out_ref):
    a = _ALPHA
    agg = agg_ref[...]
    s = jnp.sum(agg, axis=0)
    x_out_ref[...] = (1.0 - a) * (x_ref[...] + s) + a * h_ref[...]
    agg_out_ref[...] = (1.0 - a) * agg + a * nb_ref[...]


@jax.jit
def kernel(x, neighbor_agg, h, neighbor):
    n, d = x.shape
    k = neighbor_agg.shape[0]
    blk = _BLOCK
    grid = (n // blk,)

    row_spec = pl.BlockSpec((blk, d), lambda i: (i, 0))
    hop_spec = pl.BlockSpec((k, blk, d), lambda i: (0, i, 0))

    return pl.pallas_call(
        _appnp_block,
        grid=grid,
        in_specs=[row_spec, hop_spec, row_spec, hop_spec],
        out_specs=[row_spec, hop_spec],
        out_shape=[
            jax.ShapeDtypeStruct((n, d), x.dtype),
            jax.ShapeDtypeStruct((k, n, d), neighbor_agg.dtype),
        ],
        compiler_params=pltpu.CompilerParams(
            dimension_semantics=("arbitrary",),
        ),
    )(x, neighbor_agg, h, neighbor)


# speedup vs baseline: 1.4021x; 1.0235x over previous
import functools

import jax
import jax.numpy as jnp
from jax.experimental import pallas as pl
from jax.experimental.pallas import tpu as pltpu

_ALPHA = 0.1
_BLOCK = 2000


def _appnp_block(x_ref, agg_ref, h_ref, nb_ref, x_out_ref, agg_out_ref):
    a = _ALPHA
    agg = agg_ref[...]
    s = jnp.sum(agg, axis=0)
    x_out_ref[...] = (1.0 - a) * (x_ref[...] + s) + a * h_ref[...]
    agg_out_ref[...] = (1.0 - a) * agg + a * nb_ref[...]


@jax.jit
def kernel(x, neighbor_agg, h, neighbor):
    n, d = x.shape
    k = neighbor_agg.shape[0]
    blk = _BLOCK
    grid = (n // blk,)

    row_spec = pl.BlockSpec((blk, d), lambda i: (i, 0))
    hop_spec = pl.BlockSpec((k, blk, d), lambda i: (0, i, 0))

    return pl.pallas_call(
        _appnp_block,
        grid=grid,
        in_specs=[row_spec, hop_spec, row_spec, hop_spec],
        out_specs=[row_spec, hop_spec],
        out_shape=[
            jax.ShapeDtypeStruct((n, d), x.dtype),
            jax.ShapeDtypeStruct((k, n, d), neighbor_agg.dtype),
        ],
        compiler_params=pltpu.CompilerParams(
            dimension_semantics=("parallel",),
        ),
    )(x, neighbor_agg, h, neighbor)
